# baseline (speedup 1.0000x reference)
import functools

import jax
import jax.numpy as jnp
from jax import lax
from jax.experimental import pallas as pl
from jax.experimental.pallas import tpu as pltpu
from jax.experimental.pallas import tpu_sc as plsc

B = 16384
NUM_FEAT = 32
CAT_VOCAB = 1001
CAT_DIM = 16
EMB_OTHER = 64
HIDDEN = 128
PAIR = 2 * EMB_OTHER


def _sc_gather(xc0, xc1, nidc, nidm, pcd_t, ct_t, card_p, merch_p):
    info = plsc.get_sparse_core_info()
    NC, NS = info.num_cores, info.num_subcores
    NW = NC * NS
    n = B // NW
    CH = 128
    nchunk = n // CH

    mesh = plsc.VectorSubcoreMesh(core_axis_name="c", subcore_axis_name="s")

    @functools.partial(
        pl.kernel,
        mesh=mesh,
        compiler_params=pltpu.CompilerParams(needs_layout_passes=False),
        out_type=[
            jax.ShapeDtypeStruct((CAT_DIM, B), jnp.float32),
            jax.ShapeDtypeStruct((CAT_DIM, B), jnp.float32),
            jax.ShapeDtypeStruct((B, PAIR), jnp.float32),
            jax.ShapeDtypeStruct((B, PAIR), jnp.float32),
        ],
        scratch_types=[
            pltpu.VMEM((4, B // (2 * 16)), jnp.int32),
            pltpu.VMEM((CAT_DIM, CAT_VOCAB), jnp.float32),
            pltpu.VMEM((CAT_DIM, CAT_VOCAB), jnp.float32),
            pltpu.VMEM((CAT_DIM, B // (2 * 16)), jnp.float32),
            pltpu.VMEM((CAT_DIM, B // (2 * 16)), jnp.float32),
            pltpu.VMEM((2, 128, PAIR), jnp.float32),
            pltpu.VMEM((2, 128, PAIR), jnp.float32),
            pltpu.SemaphoreType.DMA,
        ],
    )
    def k(xc0_h, xc1_h, nidc_h, nidm_h, pcd_h, ct_h, card_h, merch_h,
          e0_o, e1_o, card_o, merch_o,
          idx_v, pcd_v, ct_v, e0_b, e1_b, card_b, merch_b, sem):
        wid = lax.axis_index("s") * NC + lax.axis_index("c")
        base = wid * n
        src = pl.ds(base, n)
        pltpu.sync_copy(xc0_h.at[src], idx_v.at[0])
        pltpu.sync_copy(xc1_h.at[src], idx_v.at[1])
        pltpu.sync_copy(nidc_h.at[src], idx_v.at[2])
        pltpu.sync_copy(nidm_h.at[src], idx_v.at[3])
        pltpu.sync_copy(pcd_h, pcd_v)
        pltpu.sync_copy(ct_h, ct_v)

        def drain_and_flush(c):
            s = c % 2
            pltpu.make_async_copy(card_h.at[pl.ds(0, CH), :], card_b.at[s], sem).wait()
            pltpu.make_async_copy(merch_h.at[pl.ds(0, CH), :], merch_b.at[s], sem).wait()
            out = pl.ds(base + c * CH, CH)
            pltpu.sync_copy(card_b.at[s], card_o.at[out])
            pltpu.sync_copy(merch_b.at[s], merch_o.at[out])

        for c in range(nchunk):
            s = c % 2

            def issue(g, _):
                qb = c * CH + g * 16
                vc = lax.shift_right_logical(idx_v[2, pl.ds(qb, 16)], 1)
                vm = lax.shift_right_logical(idx_v[3, pl.ds(qb, 16)], 1)
                for lane in range(16):
                    row = pl.ds(g * 16 + lane, 1)
                    pltpu.async_copy(card_h.at[pl.ds(vc[lane], 1), :], card_b.at[s, row, :], sem)
                    pltpu.async_copy(merch_h.at[pl.ds(vm[lane], 1), :], merch_b.at[s, row, :], sem)
                return _

            lax.fori_loop(0, CH // 16, issue, None)
            if c > 0:
                drain_and_flush(c - 1)

        def egroup(g, _):
            qb = g * 16
            v0 = jnp.clip(idx_v[0, pl.ds(qb, 16)] + 1, 0, CAT_VOCAB - 1)
            v1 = jnp.clip(idx_v[1, pl.ds(qb, 16)] + 1, 0, CAT_VOCAB - 1)
            for kk in range(CAT_DIM):
                kv = jnp.full((16,), kk, jnp.int32)
                e0_b[kk, pl.ds(qb, 16)] = plsc.load_gather(pcd_v, [kv, v0])
                e1_b[kk, pl.ds(qb, 16)] = plsc.load_gather(ct_v, [kv, v1])
            return _

        lax.fori_loop(0, n // 16, egroup, None)
        drain_and_flush(nchunk - 1)
        out = pl.ds(base, n)
        pltpu.sync_copy(e0_b, e0_o.at[:, out])
        pltpu.sync_copy(e1_b, e1_o.at[:, out])

    return k(xc0, xc1, nidc, nidm, pcd_t, ct_t, card_p, merch_p)


_BLK = 1024


def _dgt(a, w):
    return lax.dot_general(a, w, dimension_numbers=(((0,), (0,)), ((), ())),
                           preferred_element_type=jnp.float32)


def _tc_body(xn, e0r, e1r, cr, mr, pc, pm, wt, bt, wc, bc, wm, bm, to, co, mo):
    acc = _dgt(xn[:], wt[pl.ds(0, NUM_FEAT), :])
    acc += _dgt(e0r[:], wt[pl.ds(NUM_FEAT, CAT_DIM), :])
    acc += _dgt(e1r[:], wt[pl.ds(NUM_FEAT + CAT_DIM, CAT_DIM), :])
    to[:] = acc + bt[:]
    codd = (pc[:] & 1) == 1
    card = jnp.where(codd, cr[:, EMB_OTHER:], cr[:, :EMB_OTHER])
    modd = (pm[:] & 1) == 1
    merch = jnp.where(modd, mr[:, EMB_OTHER:], mr[:, :EMB_OTHER])
    co[:] = jnp.dot(card, wc[:], preferred_element_type=jnp.float32) + bc[:]
    mo[:] = jnp.dot(merch, wm[:], preferred_element_type=jnp.float32) + bm[:]


def _tc_forward(x_num_t, e0_t, e1_t, card_p, merch_p, nidc_col, nidm_col,
                W_trans, b_trans, W_card, b_card, W_merch, b_merch):
    grid = (B // _BLK,)
    col_blk = lambda h: pl.BlockSpec((h, _BLK), lambda i: (0, i))
    row_blk = lambda w: pl.BlockSpec((_BLK, w), lambda i: (i, 0))
    full = lambda a: pl.BlockSpec(a.shape, lambda i: (0,) * a.ndim)
    return pl.pallas_call(
        _tc_body,
        grid=grid,
        in_specs=[
            col_blk(NUM_FEAT), col_blk(CAT_DIM), col_blk(CAT_DIM),
            row_blk(PAIR), row_blk(PAIR), row_blk(1), row_blk(1),
            full(W_trans), full(b_trans), full(W_card), full(b_card),
            full(W_merch), full(b_merch),
        ],
        out_specs=[row_blk(HIDDEN), row_blk(HIDDEN), row_blk(HIDDEN)],
        out_shape=[jax.ShapeDtypeStruct((B, HIDDEN), jnp.float32)] * 3,
    )(x_num_t, e0_t, e1_t, card_p, merch_p, nidc_col, nidm_col,
      W_trans, b_trans, W_card, b_card, W_merch, b_merch)


def kernel(x_num, x_cat, n_id_card, n_id_merchant,
           emb_pcd, emb_ct, W_trans, b_trans,
           emb_card, W_card, b_card,
           emb_merch, W_merch, b_merch):
    xc0 = x_cat[:, 0].astype(jnp.int32)
    xc1 = x_cat[:, 1].astype(jnp.int32)
    nidc = n_id_card.astype(jnp.int32)
    nidm = n_id_merchant.astype(jnp.int32)
    card_pairs = emb_card.reshape(-1, PAIR)
    merch_pairs = emb_merch.reshape(-1, PAIR)
    e0_t, e1_t, card_p, merch_p = _sc_gather(
        xc0, xc1, nidc, nidm, emb_pcd.T, emb_ct.T, card_pairs, merch_pairs)
    b_t = b_trans.reshape(1, HIDDEN)
    b_c = b_card.reshape(1, HIDDEN)
    b_m = b_merch.reshape(1, HIDDEN)
    return _tc_forward(x_num.T, e0_t, e1_t, card_p, merch_p,
                       nidc.reshape(B, 1), nidm.reshape(B, 1),
                       W_trans, b_t, W_card, b_c, W_merch, b_m)
